# Initial kernel scaffold; baseline (speedup 1.0000x reference)
import jax
import jax.numpy as jnp
from jax.experimental import pallas as pl


def kernel(hidden_states, gate_w, w1, w2):
    raise NotImplementedError("write your pallas kernel here")


# speedup vs baseline: 1.2875x; 1.2875x over previous
import functools

import jax
import jax.numpy as jnp
from jax.experimental import pallas as pl

HIDDEN = 1024
FFN = 1024
NUM_EXPERTS = 64
TOP_K = 2
TOKENS = 128


def _router_kernel(x_ref, gw_ref, logits_ref, comb_ref):
    x = x_ref[...]
    gw = gw_ref[...]
    logits = jax.lax.dot_general(
        x, gw, (((1,), (0,)), ((), ())), preferred_element_type=jnp.float32
    )
    logits_ref[...] = logits
    m = jnp.max(logits, axis=-1, keepdims=True)
    ex = jnp.exp(logits - m)
    probs = ex / jnp.sum(ex, axis=-1, keepdims=True)
    col = jax.lax.broadcasted_iota(jnp.int32, probs.shape, 1)
    big = jnp.int32(NUM_EXPERTS)
    m1 = jnp.max(probs, axis=-1, keepdims=True)
    i1 = jnp.min(jnp.where(probs == m1, col, big), axis=-1, keepdims=True)
    oh1 = col == i1
    probs2 = jnp.where(oh1, -1.0, probs)
    m2 = jnp.max(probs2, axis=-1, keepdims=True)
    i2 = jnp.min(jnp.where(probs2 == m2, col, big), axis=-1, keepdims=True)
    oh2 = col == i2
    denom = m1 + m2
    comb_ref[...] = (jnp.where(oh1, m1, 0.0) + jnp.where(oh2, m2, 0.0)) / denom


def _expert_kernel(x_ref, w1_ref, w2_ref, comb_ref, out_ref):
    e = pl.program_id(0)

    @pl.when(e == 0)
    def _():
        out_ref[...] = jnp.zeros_like(out_ref)

    x = x_ref[...]
    w1e = w1_ref[0]
    h = jax.lax.dot_general(
        x, w1e, (((1,), (1,)), ((), ())), preferred_element_type=jnp.float32
    )
    gate = h[:, :FFN]
    up = h[:, FFN:]
    act = gate * jax.lax.logistic(gate) * up
    w2e = w2_ref[0]
    eo = jax.lax.dot_general(
        act, w2e, (((1,), (1,)), ((), ())), preferred_element_type=jnp.float32
    )
    comb = comb_ref[...]
    col = jax.lax.broadcasted_iota(jnp.int32, comb.shape, 1)
    cw = jnp.sum(jnp.where(col == e, comb, 0.0), axis=1, keepdims=True)
    out_ref[...] += cw * eo


@jax.jit
def kernel(hidden_states, gate_w, w1, w2):
    logits, comb = pl.pallas_call(
        _router_kernel,
        out_shape=(
            jax.ShapeDtypeStruct((TOKENS, NUM_EXPERTS), jnp.float32),
            jax.ShapeDtypeStruct((TOKENS, NUM_EXPERTS), jnp.float32),
        ),
    )(hidden_states, gate_w)

    out = pl.pallas_call(
        _expert_kernel,
        grid=(NUM_EXPERTS,),
        in_specs=[
            pl.BlockSpec((TOKENS, HIDDEN), lambda e: (0, 0)),
            pl.BlockSpec((1, 2 * FFN, HIDDEN), lambda e: (e, 0, 0)),
            pl.BlockSpec((1, HIDDEN, FFN), lambda e: (e, 0, 0)),
            pl.BlockSpec((TOKENS, NUM_EXPERTS), lambda e: (0, 0)),
        ],
        out_specs=pl.BlockSpec((TOKENS, HIDDEN), lambda e: (0, 0)),
        out_shape=jax.ShapeDtypeStruct((TOKENS, HIDDEN), jnp.float32),
    )(hidden_states, w1, w2, comb)

    return (out, logits)
